# baseline (speedup 1.0000x reference)
import functools

import jax
import jax.numpy as jnp
from jax import lax
from jax.experimental import pallas as pl
from jax.experimental.pallas import tpu as pltpu

B, S, H = 4, 8192, 1024
ROWS = B * S
MAX_POS = 8192
TK = 256
RING = 32


def _tc_body(idx_ref, x_ref, tab_ref, out_ref, emb, sem):
    blk = pl.program_id(0)

    def issue(r, carry):
        j = idx_ref[blk * TK + r]
        pltpu.async_copy(tab_ref.at[pl.ds(j, 1), :], emb.at[pl.ds(r, 1), :],
                         sem)
        return carry

    def issue_after_wait(r, carry):
        pltpu.make_async_copy(tab_ref.at[pl.ds(0, 1), :],
                              emb.at[pl.ds(r - RING, 1), :], sem).wait()
        return issue(r, carry)

    lax.fori_loop(0, RING, issue, 0)
    lax.fori_loop(RING, TK, issue_after_wait, 0)

    def drain(r, carry):
        pltpu.make_async_copy(tab_ref.at[pl.ds(0, 1), :],
                              emb.at[pl.ds(r, 1), :], sem).wait()
        return carry

    lax.fori_loop(TK - RING, TK, drain, 0)
    out_ref[...] = x_ref[...] + emb[...]


@jax.jit
def kernel(x, input_pos, position_embeddings):
    x2 = x.reshape(ROWS, H)
    pos = input_pos.reshape(ROWS).astype(jnp.int32)
    grid_spec = pltpu.PrefetchScalarGridSpec(
        num_scalar_prefetch=1,
        grid=(ROWS // TK,),
        in_specs=[
            pl.BlockSpec((TK, H), lambda i, idx_ref: (i, 0)),
            pl.BlockSpec(memory_space=pl.ANY),
        ],
        out_specs=pl.BlockSpec((TK, H), lambda i, idx_ref: (i, 0)),
        scratch_shapes=[
            pltpu.VMEM((TK, H), jnp.float32),
            pltpu.SemaphoreType.DMA,
        ],
    )
    out = pl.pallas_call(
        _tc_body,
        grid_spec=grid_spec,
        out_shape=jax.ShapeDtypeStruct((ROWS, H), jnp.float32),
    )(pos, x2, position_embeddings)
    return out.reshape(B, S, H)


# speedup vs baseline: 2.7937x; 2.7937x over previous
import functools

import jax
import jax.numpy as jnp
from jax import lax
from jax.experimental import pallas as pl
from jax.experimental.pallas import tpu as pltpu

B, S, H = 4, 8192, 1024
ROWS = B * S
MAX_POS = 8192
TK = 256
RING = 32


def _tc_body(idx_ref, x_ref, tab_ref, out_ref, emb, sem):
    blk = pl.program_id(0)

    def issue(r, carry):
        j = idx_ref[blk * TK + r]
        pltpu.async_copy(tab_ref.at[pl.ds(j, 1), :], emb.at[pl.ds(r, 1), :],
                         sem)
        return carry

    lax.fori_loop(0, TK, issue, 0, unroll=8)
    pltpu.make_async_copy(tab_ref.at[pl.ds(0, TK), :], emb, sem).wait()
    out_ref[...] = x_ref[...] + emb[...]


@jax.jit
def kernel(x, input_pos, position_embeddings):
    x2 = x.reshape(ROWS, H)
    pos = input_pos.reshape(ROWS).astype(jnp.int32)
    grid_spec = pltpu.PrefetchScalarGridSpec(
        num_scalar_prefetch=1,
        grid=(ROWS // TK,),
        in_specs=[
            pl.BlockSpec((TK, H), lambda i, idx_ref: (i, 0)),
            pl.BlockSpec(memory_space=pl.ANY),
        ],
        out_specs=pl.BlockSpec((TK, H), lambda i, idx_ref: (i, 0)),
        scratch_shapes=[
            pltpu.VMEM((TK, H), jnp.float32),
            pltpu.SemaphoreType.DMA,
        ],
    )
    out = pl.pallas_call(
        _tc_body,
        grid_spec=grid_spec,
        out_shape=jax.ShapeDtypeStruct((ROWS, H), jnp.float32),
    )(pos, x2, position_embeddings)
    return out.reshape(B, S, H)


# speedup vs baseline: 4.4645x; 1.5981x over previous
import functools

import jax
import jax.numpy as jnp
from jax import lax
from jax.experimental import pallas as pl
from jax.experimental.pallas import tpu as pltpu
from jax.experimental.pallas import tpu_sc as plsc

B, S, H = 4, 8192, 1024
ROWS = B * S
MAX_POS = 8192

TK = 256
R_TC = 43 * TK
R_SC = ROWS - R_TC

NC, NS, L = 2, 16, 16
NW = NC * NS
ROWS_PER_W = R_SC // NW
CHUNK = 8
NCHUNKS = ROWS_PER_W // CHUNK
NB = 7
D = 5
VREGS_PER_ROW = H // L


def _sc_body(x_hbm, pos_hbm, tab_hbm, out_hbm, idx_v, ebuf, xobuf, *sems):
    gsems = sems[:NB]
    xsems = sems[NB:2 * NB]
    osems = sems[2 * NB:]
    wid = lax.axis_index("s") * NC + lax.axis_index("c")
    base = wid * ROWS_PER_W
    pltpu.sync_copy(pos_hbm.at[pl.ds(base, ROWS_PER_W)], idx_v)

    def fetch(n, b):
        idx_slice = idx_v.at[pl.ds(n * CHUNK, CHUNK)]
        pltpu.async_copy(tab_hbm.at[idx_slice], ebuf.at[b], gsems[b])
        pltpu.async_copy(x_hbm.at[pl.ds(base + n * CHUNK, CHUNK)],
                         xobuf.at[b], xsems[b])

    for n in range(D):
        fetch(n, n)

    T_OUTER = (NCHUNKS + NB - 1) // NB

    def outer(t, carry):
        for b in range(NB):
            s = t * NB + b

            n = s + D
            b_n = (b + D) % NB

            @pl.when(n < NCHUNKS)
            def _():
                @pl.when(n >= NB)
                def _():
                    pltpu.make_async_copy(
                        xobuf.at[b_n],
                        out_hbm.at[pl.ds(base + (n - NB) * CHUNK, CHUNK)],
                        osems[b_n]).wait()

                fetch(n, b_n)

            @pl.when(s < NCHUNKS)
            def _():
                rb = base + s * CHUNK
                idx_slice = idx_v.at[pl.ds(s * CHUNK, CHUNK)]
                pltpu.make_async_copy(tab_hbm.at[idx_slice], ebuf.at[b],
                                      gsems[b]).wait()
                pltpu.make_async_copy(x_hbm.at[pl.ds(rb, CHUNK)], xobuf.at[b],
                                      xsems[b]).wait()

                def row_body(r, c2):
                    for c in range(VREGS_PER_ROW):
                        sl = pl.ds(c * L, L)
                        plsc.addupdate(xobuf.at[b, r, sl], ebuf[b, r, sl])
                    return c2

                lax.fori_loop(0, CHUNK, row_body, 0)
                pltpu.async_copy(xobuf.at[b], out_hbm.at[pl.ds(rb, CHUNK)],
                                 osems[b])
        return carry

    lax.fori_loop(0, T_OUTER, outer, 0)

    for c in range(NCHUNKS - NB, NCHUNKS):
        b = c % NB
        pltpu.make_async_copy(xobuf.at[b],
                              out_hbm.at[pl.ds(base + c * CHUNK, CHUNK)],
                              osems[b]).wait()


def _run_sc(x2, pos, table):
    run = functools.partial(
        pl.kernel,
        out_type=jax.ShapeDtypeStruct((ROWS, H), jnp.float32),
        mesh=plsc.VectorSubcoreMesh(core_axis_name="c", subcore_axis_name="s"),
        scratch_types=[
            pltpu.VMEM((ROWS_PER_W,), jnp.int32),
            pltpu.VMEM((NB, CHUNK, H), jnp.float32),
            pltpu.VMEM((NB, CHUNK, H), jnp.float32),
        ] + [pltpu.SemaphoreType.DMA] * (3 * NB),
    )(_sc_body)
    return run(x2, pos, table)


def _tc_body(idx_ref, x_ref, tab_ref, out_ref, emb, sem):
    blk = pl.program_id(0)

    def issue(r, carry):
        j = idx_ref[R_SC + blk * TK + r]
        pltpu.async_copy(tab_ref.at[pl.ds(j, 1), :], emb.at[pl.ds(r, 1), :],
                         sem)
        return carry

    lax.fori_loop(0, TK, issue, 0, unroll=8)
    pltpu.make_async_copy(tab_ref.at[pl.ds(0, TK), :], emb, sem).wait()
    out_ref[...] = x_ref[...] + emb[...]


def _run_tc(x2, pos, table):
    grid_spec = pltpu.PrefetchScalarGridSpec(
        num_scalar_prefetch=1,
        grid=(R_TC // TK,),
        in_specs=[
            pl.BlockSpec((TK, H), lambda i, idx_ref: (i + R_SC // TK, 0)),
            pl.BlockSpec(memory_space=pl.ANY),
        ],
        out_specs=pl.BlockSpec((TK, H), lambda i, idx_ref: (i, 0)),
        scratch_shapes=[
            pltpu.VMEM((TK, H), jnp.float32),
            pltpu.SemaphoreType.DMA,
        ],
    )
    return pl.pallas_call(
        _tc_body,
        grid_spec=grid_spec,
        out_shape=jax.ShapeDtypeStruct((R_TC, H), jnp.float32),
    )(pos, x2, table)


@jax.jit
def kernel(x, input_pos, position_embeddings):
    x2 = x.reshape(ROWS, H)
    pos = input_pos.reshape(ROWS).astype(jnp.int32)
    sc_out = _run_sc(x2, pos, position_embeddings)
    tc_out = _run_tc(x2, pos, position_embeddings)
    out = lax.dynamic_update_slice(sc_out, tc_out, (R_SC, 0))
    return out.reshape(B, S, H)
